# Initial kernel scaffold; baseline (speedup 1.0000x reference)
import jax
import jax.numpy as jnp
from jax.experimental import pallas as pl


def kernel(x, edge_index, W1, b1, W2, b2, W3, b3):
    raise NotImplementedError("write your pallas kernel here")


# speedup vs baseline: 160.1759x; 160.1759x over previous
import functools

import jax
import jax.numpy as jnp
from jax import lax
from jax.experimental import pallas as pl
from jax.experimental.pallas import tpu as pltpu
from jax.experimental.pallas import tpu_sc as plsc

NC = 2
NS = 16
NW = NC * NS
LANES = 16

K_EDGE = 125


def _z():
  return jnp.int32(0)


def _sc_mesh():
  return plsc.VectorSubcoreMesh(core_axis_name="c", subcore_axis_name="s")


def _degree_kernel(src2d, dst2d, n_nodes, e_per_tile):
  @functools.partial(
      pl.kernel,
      out_type=jax.ShapeDtypeStruct((NW, 2 * n_nodes), jnp.float32),
      mesh=_sc_mesh(),
      compiler_params=pltpu.CompilerParams(needs_layout_passes=False,
                                           use_tc_tiling_on_sc=False),
      scratch_types=[
          pltpu.VMEM((e_per_tile,), jnp.int32),
          pltpu.VMEM((e_per_tile,), jnp.int32),
          pltpu.VMEM((2 * n_nodes,), jnp.float32),
      ],
  )
  def deg_k(src_hbm, dst_hbm, out_hbm, sidx, didx, hist):
    c = lax.axis_index("c")
    s = lax.axis_index("s")
    wid = c * jnp.int32(NS) + s
    pltpu.sync_copy(src_hbm.at[wid], sidx)
    pltpu.sync_copy(dst_hbm.at[wid], didx)

    zeros = jnp.zeros((LANES,), jnp.float32)
    L = jnp.int32(LANES)

    def zbody(i, carry):
      hist[pl.ds(i * L, LANES)] = zeros
      return carry

    lax.fori_loop(jnp.int32(0), jnp.int32(2 * n_nodes // LANES), zbody,
                  jnp.int32(0))

    ones = jnp.ones((LANES,), jnp.float32)
    two = jnp.int32(2)
    one = jnp.int32(1)

    def body(i, carry):
      si = sidx[pl.ds(i * L, LANES)]
      di = didx[pl.ds(i * L, LANES)]
      plsc.addupdate_scatter(hist, [si * two], ones)
      plsc.addupdate_scatter(hist, [di * two + one], ones)
      return carry

    lax.fori_loop(jnp.int32(0), jnp.int32(e_per_tile // LANES), body,
                  jnp.int32(0))

    pltpu.sync_copy(hist, out_hbm.at[wid])

  return deg_k(src2d, dst2d)


def _aggregate_kernel(h, src3d, dst3d, n_nodes, nblk, feat):
  rows_per_tile = n_nodes // NS

  @functools.partial(
      pl.kernel,
      out_type=jax.ShapeDtypeStruct((NC, n_nodes, feat), jnp.float32),
      mesh=_sc_mesh(),
      compiler_params=pltpu.CompilerParams(use_tc_tiling_on_sc=False),
      scratch_types=[
          pltpu.VMEM_SHARED((n_nodes, feat), jnp.float32),
          pltpu.VMEM((nblk, K_EDGE), jnp.int32),
          pltpu.VMEM((nblk, K_EDGE), jnp.int32),
          pltpu.VMEM((K_EDGE, feat), jnp.float32),
          pltpu.VMEM((16, feat), jnp.float32),
          pltpu.SemaphoreType.DMA,
      ],
  )
  def agg_k(h_hbm, src_hbm, dst_hbm, out_hbm, acc_sh, sidx, didx, rows,
            zbuf, sem):
    c = lax.axis_index("c")
    s = lax.axis_index("s")
    wid = c * jnp.int32(NS) + s
    pltpu.sync_copy(src_hbm.at[wid], sidx)
    pltpu.sync_copy(dst_hbm.at[wid], didx)

    zeros = jnp.zeros((LANES,), jnp.float32)
    vpr = jnp.int32(feat // LANES)
    nfull = NS - 1
    crows = (n_nodes // NS // 8) * 8
    lrows = n_nodes - nfull * crows
    cbase = pl.multiple_of(s * jnp.int32(crows), 8)

    def zv(i, carry):
      zbuf[i // vpr, pl.ds((i % vpr) * jnp.int32(LANES), LANES)] = zeros
      return carry

    lax.fori_loop(jnp.int32(0), jnp.int32(16 * (feat // LANES)), zv,
                  jnp.int32(0))

    def zc(m, carry):
      pltpu.sync_copy(zbuf, acc_sh.at[pl.ds(cbase + m * jnp.int32(16), 16)])
      return carry

    nz = jnp.where(s == jnp.int32(nfull), jnp.int32(lrows // 16),
                   jnp.int32(crows // 16))
    lax.fori_loop(jnp.int32(0), nz, zc, jnp.int32(0))
    plsc.subcore_barrier()

    def body(j, carry):
      pltpu.async_copy(h_hbm.at[sidx.at[j]], rows, sem).wait()
      pltpu.sync_copy(rows, acc_sh.at[didx.at[j]], add=True)
      return carry

    lax.fori_loop(jnp.int32(0), jnp.int32(nblk), body, jnp.int32(0))
    plsc.subcore_barrier()

    @pl.when(s < jnp.int32(nfull))
    def _():
      pltpu.sync_copy(acc_sh.at[pl.ds(cbase, crows)],
                      out_hbm.at[c, pl.ds(cbase, crows)])

    @pl.when(s == jnp.int32(nfull))
    def _():
      lbase = pl.multiple_of(jnp.int32(nfull * crows), 8)
      pltpu.sync_copy(acc_sh.at[pl.ds(lbase, lrows)],
                      out_hbm.at[c, pl.ds(lbase, lrows)])

  return agg_k(h, src3d, dst3d)


def _tc1(hist, x, w1, n_nodes, feat, br):
  def body(hist_ref, x_ref, w_ref, h_ref, inv_ref):
    cnt = jnp.sum(hist_ref[...], axis=0) + 1.0
    inv = lax.rsqrt(cnt)
    inv_ref[...] = inv
    h = jnp.dot(x_ref[...], w_ref[...], preferred_element_type=jnp.float32)
    h_ref[...] = h * inv[:, 0:1]

  grid = n_nodes // br
  return pl.pallas_call(
      body,
      grid=(grid,),
      in_specs=[
          pl.BlockSpec((NW, br, 2), lambda i: (_z(), i, _z())),
          pl.BlockSpec((br, feat), lambda i: (i, _z())),
          pl.BlockSpec((feat, feat), lambda i: (_z(), _z())),
      ],
      out_specs=[
          pl.BlockSpec((br, feat), lambda i: (i, _z())),
          pl.BlockSpec((br, 2), lambda i: (i, _z())),
      ],
      out_shape=[
          jax.ShapeDtypeStruct((n_nodes, feat), jnp.float32),
          jax.ShapeDtypeStruct((n_nodes, 2), jnp.float32),
      ],
  )(hist, x, w1)


def _tc_mid(part, hself, inv, w, b, n_nodes, feat, br):
  def body(p_ref, h_ref, inv_ref, w_ref, b_ref, o_ref):
    agg = p_ref[0] + p_ref[1] + h_ref[...]
    inv = inv_ref[...]
    agg = agg * inv[:, 1:2] + b_ref[...]
    hrelu = jnp.maximum(agg, 0.0)
    o = jnp.dot(hrelu, w_ref[...], preferred_element_type=jnp.float32)
    o_ref[...] = o * inv[:, 0:1]

  grid = n_nodes // br
  return pl.pallas_call(
      body,
      grid=(grid,),
      in_specs=[
          pl.BlockSpec((NC, br, feat), lambda i: (_z(), i, _z())),
          pl.BlockSpec((br, feat), lambda i: (i, _z())),
          pl.BlockSpec((br, 2), lambda i: (i, _z())),
          pl.BlockSpec((feat, feat), lambda i: (_z(), _z())),
          pl.BlockSpec((1, feat), lambda i: (_z(), _z())),
      ],
      out_specs=pl.BlockSpec((br, feat), lambda i: (i, _z())),
      out_shape=jax.ShapeDtypeStruct((n_nodes, feat), jnp.float32),
  )(part, hself, inv, w, b)


def _tc_out(part, hself, inv, w3, b2, b3, n_nodes, feat, ncls, br):
  def body(q_ref, h_ref, inv_ref, w_ref, b2_ref, b3_ref, o_ref):
    agg = q_ref[0] + q_ref[1] + h_ref[...]
    agg = agg * inv_ref[...][:, 1:2] + b2_ref[...]
    hrelu = jnp.maximum(agg, 0.0)
    logits = jnp.dot(hrelu, w_ref[...], preferred_element_type=jnp.float32)
    logits = logits + b3_ref[...]
    m = jnp.max(logits, axis=1, keepdims=True)
    e = jnp.exp(logits - m)
    lse = jnp.log(jnp.sum(e, axis=1, keepdims=True)) + m
    o_ref[...] = logits - lse

  grid = n_nodes // br
  return pl.pallas_call(
      body,
      grid=(grid,),
      in_specs=[
          pl.BlockSpec((NC, br, feat), lambda i: (_z(), i, _z())),
          pl.BlockSpec((br, feat), lambda i: (i, _z())),
          pl.BlockSpec((br, 2), lambda i: (i, _z())),
          pl.BlockSpec((feat, ncls), lambda i: (_z(), _z())),
          pl.BlockSpec((1, feat), lambda i: (_z(), _z())),
          pl.BlockSpec((1, ncls), lambda i: (_z(), _z())),
      ],
      out_specs=pl.BlockSpec((br, ncls), lambda i: (i, _z())),
      out_shape=jax.ShapeDtypeStruct((n_nodes, ncls), jnp.float32),
  )(part, hself, inv, w3, b2, b3)


def kernel(x, edge_index, W1, b1, W2, b2, W3, b3):
  n_nodes, feat = x.shape
  n_edges = edge_index.shape[1]
  ncls = W3.shape[1]
  e_per_tile = n_edges // NW
  nblk = e_per_tile // K_EDGE
  assert e_per_tile * NW == n_edges and nblk * K_EDGE == e_per_tile
  assert n_nodes % (NS * 125) == 0 and feat % LANES == 0

  x = x.astype(jnp.float32)
  src = edge_index[0].astype(jnp.int32)
  dst = edge_index[1].astype(jnp.int32)
  src2d = src.reshape(NW, e_per_tile)
  dst2d = dst.reshape(NW, e_per_tile)
  src3d = src.reshape(NW, nblk, K_EDGE)
  dst3d = dst.reshape(NW, nblk, K_EDGE)
  b1r = b1.astype(jnp.float32).reshape(1, feat)
  b2r = b2.astype(jnp.float32).reshape(1, feat)
  b3r = b3.astype(jnp.float32).reshape(1, ncls)

  hist = _degree_kernel(src2d, dst2d, n_nodes, e_per_tile)
  hist = hist.reshape(NW, n_nodes, 2)
  h1, inv = _tc1(hist, x, W1.astype(jnp.float32), n_nodes, feat, 1000)
  p1 = _aggregate_kernel(h1, src3d, dst3d, n_nodes, nblk, feat)
  t2 = _tc_mid(p1, h1, inv, W2.astype(jnp.float32), b1r, n_nodes, feat, 1000)
  p2 = _aggregate_kernel(t2, src3d, dst3d, n_nodes, nblk, feat)
  out = _tc_out(p2, t2, inv, W3.astype(jnp.float32), b2r, b3r,
                n_nodes, feat, ncls, 1000)
  return out.astype(jnp.float64)


# speedup vs baseline: 203.3448x; 1.2695x over previous
import functools

import jax
import jax.numpy as jnp
from jax import lax
from jax.experimental import pallas as pl
from jax.experimental.pallas import tpu as pltpu
from jax.experimental.pallas import tpu_sc as plsc

NC = 2
NS = 16
NW = NC * NS
LANES = 16

K_EDGE = 125
NPAD = 10240


def _z():
  return jnp.int32(0)


def _sc_mesh():
  return plsc.VectorSubcoreMesh(core_axis_name="c", subcore_axis_name="s")


def _degree_kernel(src2d, dst2d, n_nodes, e_per_tile):
  @functools.partial(
      pl.kernel,
      out_type=jax.ShapeDtypeStruct((NW, 2 * NPAD), jnp.float32),
      mesh=_sc_mesh(),
      compiler_params=pltpu.CompilerParams(needs_layout_passes=False,
                                           use_tc_tiling_on_sc=False),
      scratch_types=[
          pltpu.VMEM((e_per_tile,), jnp.int32),
          pltpu.VMEM((e_per_tile,), jnp.int32),
          pltpu.VMEM((2 * NPAD,), jnp.float32),
      ],
  )
  def deg_k(src_hbm, dst_hbm, out_hbm, sidx, didx, hist):
    c = lax.axis_index("c")
    s = lax.axis_index("s")
    wid = c * jnp.int32(NS) + s
    pltpu.sync_copy(src_hbm.at[wid], sidx)
    pltpu.sync_copy(dst_hbm.at[wid], didx)

    zeros = jnp.zeros((LANES,), jnp.float32)
    L = jnp.int32(LANES)

    def zbody(i, carry):
      hist[pl.ds(i * L, LANES)] = zeros
      return carry

    lax.fori_loop(jnp.int32(0), jnp.int32(2 * NPAD // LANES), zbody,
                  jnp.int32(0))

    ones = jnp.ones((LANES,), jnp.float32)
    noff = jnp.int32(NPAD)

    def body(i, carry):
      si = sidx[pl.ds(i * L, LANES)]
      di = didx[pl.ds(i * L, LANES)]
      plsc.addupdate_scatter(hist, [si], ones)
      plsc.addupdate_scatter(hist, [di + noff], ones)
      return carry

    lax.fori_loop(jnp.int32(0), jnp.int32(e_per_tile // LANES), body,
                  jnp.int32(0))

    pltpu.sync_copy(hist, out_hbm.at[wid])

  return deg_k(src2d, dst2d)


def _norm_kernel(hist):
  def body(hist_ref, inv_ref):
    h = hist_ref[...]
    cs = jnp.sum(h[:, 0:NPAD], axis=0) + 1.0
    cd = jnp.sum(h[:, NPAD:2 * NPAD], axis=0) + 1.0
    inv2 = lax.rsqrt(jnp.stack([cs, cd]))
    inv_ref[...] = inv2.T

  return pl.pallas_call(
      body,
      out_shape=jax.ShapeDtypeStruct((NPAD, 2), jnp.float32),
  )(hist)


def _aggregate_kernel(h, src3d, dst3d, n_nodes, nblk, feat):
  @functools.partial(
      pl.kernel,
      out_type=jax.ShapeDtypeStruct((NC, n_nodes, feat), jnp.float32),
      mesh=_sc_mesh(),
      compiler_params=pltpu.CompilerParams(use_tc_tiling_on_sc=False),
      scratch_types=[
          pltpu.VMEM_SHARED((n_nodes, feat), jnp.float32),
          pltpu.VMEM((nblk, K_EDGE), jnp.int32),
          pltpu.VMEM((nblk, K_EDGE), jnp.int32),
          pltpu.VMEM((K_EDGE, feat), jnp.float32),
          pltpu.VMEM((16, feat), jnp.float32),
          pltpu.SemaphoreType.DMA,
      ],
  )
  def agg_k(h_hbm, src_hbm, dst_hbm, out_hbm, acc_sh, sidx, didx, rows,
            zbuf, sem):
    c = lax.axis_index("c")
    s = lax.axis_index("s")
    wid = c * jnp.int32(NS) + s
    pltpu.sync_copy(src_hbm.at[wid], sidx)
    pltpu.sync_copy(dst_hbm.at[wid], didx)

    zeros = jnp.zeros((LANES,), jnp.float32)
    vpr = jnp.int32(feat // LANES)
    nfull = NS - 1
    crows = (n_nodes // NS // 8) * 8
    lrows = n_nodes - nfull * crows
    cbase = pl.multiple_of(s * jnp.int32(crows), 8)

    def zv(i, carry):
      zbuf[i // vpr, pl.ds((i % vpr) * jnp.int32(LANES), LANES)] = zeros
      return carry

    lax.fori_loop(jnp.int32(0), jnp.int32(16 * (feat // LANES)), zv,
                  jnp.int32(0))

    def zc(m, carry):
      pltpu.sync_copy(zbuf, acc_sh.at[pl.ds(cbase + m * jnp.int32(16), 16)])
      return carry

    nz = jnp.where(s == jnp.int32(nfull), jnp.int32(lrows // 16),
                   jnp.int32(crows // 16))
    lax.fori_loop(jnp.int32(0), nz, zc, jnp.int32(0))
    plsc.subcore_barrier()

    def body(j, carry):
      pltpu.async_copy(h_hbm.at[sidx.at[j]], rows, sem).wait()
      pltpu.sync_copy(rows, acc_sh.at[didx.at[j]], add=True)
      return carry

    lax.fori_loop(jnp.int32(0), jnp.int32(nblk), body, jnp.int32(0))
    plsc.subcore_barrier()

    @pl.when(s < jnp.int32(nfull))
    def _():
      pltpu.sync_copy(acc_sh.at[pl.ds(cbase, crows)],
                      out_hbm.at[c, pl.ds(cbase, crows)])

    @pl.when(s == jnp.int32(nfull))
    def _():
      lbase = pl.multiple_of(jnp.int32(nfull * crows), 8)
      pltpu.sync_copy(acc_sh.at[pl.ds(lbase, lrows)],
                      out_hbm.at[c, pl.ds(lbase, lrows)])

  return agg_k(h, src3d, dst3d)


def _tc1(inv, x, w1, n_nodes, feat, br):
  def body(inv_ref, x_ref, w_ref, h_ref):
    h = jnp.dot(x_ref[...], w_ref[...], preferred_element_type=jnp.float32)
    h_ref[...] = h * inv_ref[...][:, 0:1]

  grid = n_nodes // br
  return pl.pallas_call(
      body,
      grid=(grid,),
      in_specs=[
          pl.BlockSpec((br, 2), lambda i: (i, _z())),
          pl.BlockSpec((br, feat), lambda i: (i, _z())),
          pl.BlockSpec((feat, feat), lambda i: (_z(), _z())),
      ],
      out_specs=pl.BlockSpec((br, feat), lambda i: (i, _z())),
      out_shape=jax.ShapeDtypeStruct((n_nodes, feat), jnp.float32),
  )(inv, x, w1)


def _tc_mid(part, hself, inv, w, b, n_nodes, feat, br):
  def body(p_ref, h_ref, inv_ref, w_ref, b_ref, o_ref):
    iv = inv_ref[...]
    agg = p_ref[0] + p_ref[1] + h_ref[...]
    agg = agg * iv[:, 1:2] + b_ref[...]
    hrelu = jnp.maximum(agg, 0.0)
    o = jnp.dot(hrelu, w_ref[...], preferred_element_type=jnp.float32)
    o_ref[...] = o * iv[:, 0:1]

  grid = n_nodes // br
  return pl.pallas_call(
      body,
      grid=(grid,),
      in_specs=[
          pl.BlockSpec((NC, br, feat), lambda i: (_z(), i, _z())),
          pl.BlockSpec((br, feat), lambda i: (i, _z())),
          pl.BlockSpec((br, 2), lambda i: (i, _z())),
          pl.BlockSpec((feat, feat), lambda i: (_z(), _z())),
          pl.BlockSpec((1, feat), lambda i: (_z(), _z())),
      ],
      out_specs=pl.BlockSpec((br, feat), lambda i: (i, _z())),
      out_shape=jax.ShapeDtypeStruct((n_nodes, feat), jnp.float32),
  )(part, hself, inv, w, b)


def _tc_out(part, hself, inv, w3, b2, b3, n_nodes, feat, ncls, br):
  def body(q_ref, h_ref, inv_ref, w_ref, b2_ref, b3_ref, o_ref):
    agg = q_ref[0] + q_ref[1] + h_ref[...]
    agg = agg * inv_ref[...][:, 1:2] + b2_ref[...]
    hrelu = jnp.maximum(agg, 0.0)
    logits = jnp.dot(hrelu, w_ref[...], preferred_element_type=jnp.float32)
    logits = logits + b3_ref[...]
    m = jnp.max(logits, axis=1, keepdims=True)
    e = jnp.exp(logits - m)
    lse = jnp.log(jnp.sum(e, axis=1, keepdims=True)) + m
    o_ref[...] = logits - lse

  grid = n_nodes // br
  return pl.pallas_call(
      body,
      grid=(grid,),
      in_specs=[
          pl.BlockSpec((NC, br, feat), lambda i: (_z(), i, _z())),
          pl.BlockSpec((br, feat), lambda i: (i, _z())),
          pl.BlockSpec((br, 2), lambda i: (i, _z())),
          pl.BlockSpec((feat, ncls), lambda i: (_z(), _z())),
          pl.BlockSpec((1, feat), lambda i: (_z(), _z())),
          pl.BlockSpec((1, ncls), lambda i: (_z(), _z())),
      ],
      out_specs=pl.BlockSpec((br, ncls), lambda i: (i, _z())),
      out_shape=jax.ShapeDtypeStruct((n_nodes, ncls), jnp.float32),
  )(part, hself, inv, w3, b2, b3)


def kernel(x, edge_index, W1, b1, W2, b2, W3, b3):
  n_nodes, feat = x.shape
  n_edges = edge_index.shape[1]
  ncls = W3.shape[1]
  e_per_tile = n_edges // NW
  nblk = e_per_tile // K_EDGE
  assert e_per_tile * NW == n_edges and nblk * K_EDGE == e_per_tile
  assert n_nodes % NS == 0 and feat % LANES == 0 and n_nodes <= NPAD

  x = x.astype(jnp.float32)
  src = edge_index[0].astype(jnp.int32)
  dst = edge_index[1].astype(jnp.int32)
  src2d = src.reshape(NW, e_per_tile)
  dst2d = dst.reshape(NW, e_per_tile)
  src3d = src.reshape(NW, nblk, K_EDGE)
  dst3d = dst.reshape(NW, nblk, K_EDGE)
  b1r = b1.astype(jnp.float32).reshape(1, feat)
  b2r = b2.astype(jnp.float32).reshape(1, feat)
  b3r = b3.astype(jnp.float32).reshape(1, ncls)

  hist = _degree_kernel(src2d, dst2d, n_nodes, e_per_tile)
  inv = _norm_kernel(hist)
  h1 = _tc1(inv, x, W1.astype(jnp.float32), n_nodes, feat, 1000)
  p1 = _aggregate_kernel(h1, src3d, dst3d, n_nodes, nblk, feat)
  t2 = _tc_mid(p1, h1, inv, W2.astype(jnp.float32), b1r, n_nodes, feat, 1000)
  p2 = _aggregate_kernel(t2, src3d, dst3d, n_nodes, nblk, feat)
  out = _tc_out(p2, t2, inv, W3.astype(jnp.float32), b2r, b3r,
                n_nodes, feat, ncls, 1000)
  return out.astype(jnp.float64)


# speedup vs baseline: 249.3643x; 1.2263x over previous
import functools

import jax
import jax.numpy as jnp
from jax import lax
from jax.experimental import pallas as pl
from jax.experimental.pallas import tpu as pltpu
from jax.experimental.pallas import tpu_sc as plsc

NC = 2
NS = 16
NW = NC * NS
LANES = 16

K_EDGE = 125
NPAD = 10240


def _z():
  return jnp.int32(0)


def _one():
  return jnp.int32(1)


def _sc_mesh():
  return plsc.VectorSubcoreMesh(core_axis_name="c", subcore_axis_name="s")


def _degree_kernel(src2d, dst2d, n_nodes, e_per_tile):
  @functools.partial(
      pl.kernel,
      out_type=jax.ShapeDtypeStruct((NW, 2 * NPAD), jnp.float32),
      mesh=_sc_mesh(),
      compiler_params=pltpu.CompilerParams(needs_layout_passes=False,
                                           use_tc_tiling_on_sc=False),
      scratch_types=[
          pltpu.VMEM((e_per_tile,), jnp.int32),
          pltpu.VMEM((e_per_tile,), jnp.int32),
          pltpu.VMEM((2 * NPAD,), jnp.float32),
      ],
  )
  def deg_k(src_hbm, dst_hbm, out_hbm, sidx, didx, hist):
    c = lax.axis_index("c")
    s = lax.axis_index("s")
    wid = c * jnp.int32(NS) + s
    pltpu.sync_copy(src_hbm.at[wid], sidx)
    pltpu.sync_copy(dst_hbm.at[wid], didx)

    zeros = jnp.zeros((LANES,), jnp.float32)
    L = jnp.int32(LANES)

    def zbody(i, carry):
      hist[pl.ds(i * L, LANES)] = zeros
      return carry

    lax.fori_loop(jnp.int32(0), jnp.int32(2 * NPAD // LANES), zbody,
                  jnp.int32(0))

    ones = jnp.ones((LANES,), jnp.float32)
    noff = jnp.int32(NPAD)

    def body(i, carry):
      si = sidx[pl.ds(i * L, LANES)]
      di = didx[pl.ds(i * L, LANES)]
      plsc.addupdate_scatter(hist, [si], ones)
      plsc.addupdate_scatter(hist, [di + noff], ones)
      return carry

    lax.fori_loop(jnp.int32(0), jnp.int32(e_per_tile // LANES), body,
                  jnp.int32(0))

    pltpu.sync_copy(hist, out_hbm.at[wid])

  return deg_k(src2d, dst2d)


def _norm_kernel(hist):
  def body(hist_ref, inv_ref):
    h = hist_ref[...]
    cs = jnp.sum(h[:, 0:NPAD], axis=0) + 1.0
    cd = jnp.sum(h[:, NPAD:2 * NPAD], axis=0) + 1.0
    inv2 = lax.rsqrt(jnp.stack([cs, cd]))
    inv_ref[...] = inv2.T

  return pl.pallas_call(
      body,
      out_shape=jax.ShapeDtypeStruct((NPAD, 2), jnp.float32),
  )(hist)


def _aggregate_kernel(h, src3d, dst3d, n_nodes, nblk, feat):
  IB = 20
  nchunks = nblk // IB
  assert nchunks * IB == nblk and IB % 2 == 0

  @functools.partial(
      pl.kernel,
      out_type=jax.ShapeDtypeStruct((NC, n_nodes, feat), jnp.float32),
      mesh=_sc_mesh(),
      compiler_params=pltpu.CompilerParams(use_tc_tiling_on_sc=False),
      scratch_types=[
          pltpu.VMEM_SHARED((n_nodes, feat), jnp.float32),
          pltpu.VMEM((2, IB, K_EDGE), jnp.int32),
          pltpu.VMEM((2, IB, K_EDGE), jnp.int32),
          pltpu.VMEM((2, K_EDGE, feat), jnp.float32),
          pltpu.VMEM((16, feat), jnp.float32),
          pltpu.SemaphoreType.DMA,
          pltpu.SemaphoreType.DMA,
          pltpu.SemaphoreType.DMA,
      ],
  )
  def agg_k(h_hbm, src_hbm, dst_hbm, out_hbm, acc_sh, sbuf, dbuf, rows,
            zbuf, sg0, sg1, si):
    c = lax.axis_index("c")
    s = lax.axis_index("s")
    wid = c * jnp.int32(NS) + s

    pltpu.async_copy(src_hbm.at[wid, pl.ds(_z(), IB)], sbuf.at[_z()], si)
    pltpu.async_copy(dst_hbm.at[wid, pl.ds(_z(), IB)], dbuf.at[_z()], si)

    zeros = jnp.zeros((LANES,), jnp.float32)
    vpr = jnp.int32(feat // LANES)
    nfull = NS - 1
    crows = (n_nodes // NS // 8) * 8
    lrows = n_nodes - nfull * crows
    cbase = pl.multiple_of(s * jnp.int32(crows), 8)

    def zv(i, carry):
      zbuf[i // vpr, pl.ds((i % vpr) * jnp.int32(LANES), LANES)] = zeros
      return carry

    lax.fori_loop(jnp.int32(0), jnp.int32(16 * (feat // LANES)), zv,
                  jnp.int32(0))

    def zc(m, carry):
      pltpu.sync_copy(zbuf, acc_sh.at[pl.ds(cbase + m * jnp.int32(16), 16)])
      return carry

    nz = jnp.where(s == jnp.int32(nfull), jnp.int32(lrows // 16),
                   jnp.int32(crows // 16))
    lax.fori_loop(jnp.int32(0), nz, zc, jnp.int32(0))
    plsc.subcore_barrier()

    def wait_idx(t):
      pltpu.make_async_copy(src_hbm.at[wid, pl.ds(_z(), IB)],
                            sbuf.at[jnp.int32(t % 2)], si).wait()
      pltpu.make_async_copy(dst_hbm.at[wid, pl.ds(_z(), IB)],
                            dbuf.at[jnp.int32(t % 2)], si).wait()

    npairs = IB // 2
    for t in range(nchunks):
      cs = t % 2
      wait_idx(t)
      if t + 1 < nchunks:
        off = jnp.int32((t + 1) * IB)
        pltpu.async_copy(src_hbm.at[wid, pl.ds(off, IB)],
                         sbuf.at[jnp.int32((t + 1) % 2)], si)
        pltpu.async_copy(dst_hbm.at[wid, pl.ds(off, IB)],
                         dbuf.at[jnp.int32((t + 1) % 2)], si)
      sidx = sbuf.at[jnp.int32(cs)]
      didx = dbuf.at[jnp.int32(cs)]
      pltpu.async_copy(h_hbm.at[sidx.at[_z()]], rows.at[_z()], sg0)

      def pair(m, carry):
        j0 = m * jnp.int32(2)
        j1 = j0 + jnp.int32(1)
        pltpu.make_async_copy(h_hbm.at[sidx.at[j0]], rows.at[_z()], sg0).wait()
        pltpu.async_copy(h_hbm.at[sidx.at[j1]], rows.at[_one()], sg1)
        pltpu.sync_copy(rows.at[_z()], acc_sh.at[didx.at[j0]], add=True)
        pltpu.make_async_copy(h_hbm.at[sidx.at[j1]], rows.at[_one()], sg1).wait()

        @pl.when(m < jnp.int32(npairs - 1))
        def _():
          pltpu.async_copy(h_hbm.at[sidx.at[j0 + jnp.int32(2)]],
                           rows.at[_z()], sg0)

        pltpu.sync_copy(rows.at[_one()], acc_sh.at[didx.at[j1]], add=True)
        return carry

      lax.fori_loop(jnp.int32(0), jnp.int32(npairs), pair, jnp.int32(0))

    plsc.subcore_barrier()

    @pl.when(s < jnp.int32(nfull))
    def _():
      pltpu.sync_copy(acc_sh.at[pl.ds(cbase, crows)],
                      out_hbm.at[c, pl.ds(cbase, crows)])

    @pl.when(s == jnp.int32(nfull))
    def _():
      lbase = pl.multiple_of(jnp.int32(nfull * crows), 8)
      pltpu.sync_copy(acc_sh.at[pl.ds(lbase, lrows)],
                      out_hbm.at[c, pl.ds(lbase, lrows)])

  return agg_k(h, src3d, dst3d)


def _tc1(inv, x, w1, n_nodes, feat, br):
  def body(inv_ref, x_ref, w_ref, h_ref):
    h = jnp.dot(x_ref[...], w_ref[...], preferred_element_type=jnp.float32)
    h_ref[...] = h * inv_ref[...][:, 0:1]

  grid = n_nodes // br
  return pl.pallas_call(
      body,
      grid=(grid,),
      in_specs=[
          pl.BlockSpec((br, 2), lambda i: (i, _z())),
          pl.BlockSpec((br, feat), lambda i: (i, _z())),
          pl.BlockSpec((feat, feat), lambda i: (_z(), _z())),
      ],
      out_specs=pl.BlockSpec((br, feat), lambda i: (i, _z())),
      out_shape=jax.ShapeDtypeStruct((n_nodes, feat), jnp.float32),
  )(inv, x, w1)


def _tc_mid(part, hself, inv, w, b, n_nodes, feat, br):
  def body(p_ref, h_ref, inv_ref, w_ref, b_ref, o_ref):
    iv = inv_ref[...]
    agg = p_ref[0] + p_ref[1] + h_ref[...]
    agg = agg * iv[:, 1:2] + b_ref[...]
    hrelu = jnp.maximum(agg, 0.0)
    o = jnp.dot(hrelu, w_ref[...], preferred_element_type=jnp.float32)
    o_ref[...] = o * iv[:, 0:1]

  grid = n_nodes // br
  return pl.pallas_call(
      body,
      grid=(grid,),
      in_specs=[
          pl.BlockSpec((NC, br, feat), lambda i: (_z(), i, _z())),
          pl.BlockSpec((br, feat), lambda i: (i, _z())),
          pl.BlockSpec((br, 2), lambda i: (i, _z())),
          pl.BlockSpec((feat, feat), lambda i: (_z(), _z())),
          pl.BlockSpec((1, feat), lambda i: (_z(), _z())),
      ],
      out_specs=pl.BlockSpec((br, feat), lambda i: (i, _z())),
      out_shape=jax.ShapeDtypeStruct((n_nodes, feat), jnp.float32),
  )(part, hself, inv, w, b)


def _tc_out(part, hself, inv, w3, b2, b3, n_nodes, feat, ncls, br):
  def body(q_ref, h_ref, inv_ref, w_ref, b2_ref, b3_ref, o_ref):
    agg = q_ref[0] + q_ref[1] + h_ref[...]
    agg = agg * inv_ref[...][:, 1:2] + b2_ref[...]
    hrelu = jnp.maximum(agg, 0.0)
    logits = jnp.dot(hrelu, w_ref[...], preferred_element_type=jnp.float32)
    logits = logits + b3_ref[...]
    m = jnp.max(logits, axis=1, keepdims=True)
    e = jnp.exp(logits - m)
    lse = jnp.log(jnp.sum(e, axis=1, keepdims=True)) + m
    o_ref[...] = logits - lse

  grid = n_nodes // br
  return pl.pallas_call(
      body,
      grid=(grid,),
      in_specs=[
          pl.BlockSpec((NC, br, feat), lambda i: (_z(), i, _z())),
          pl.BlockSpec((br, feat), lambda i: (i, _z())),
          pl.BlockSpec((br, 2), lambda i: (i, _z())),
          pl.BlockSpec((feat, ncls), lambda i: (_z(), _z())),
          pl.BlockSpec((1, feat), lambda i: (_z(), _z())),
          pl.BlockSpec((1, ncls), lambda i: (_z(), _z())),
      ],
      out_specs=pl.BlockSpec((br, ncls), lambda i: (i, _z())),
      out_shape=jax.ShapeDtypeStruct((n_nodes, ncls), jnp.float32),
  )(part, hself, inv, w3, b2, b3)


def kernel(x, edge_index, W1, b1, W2, b2, W3, b3):
  n_nodes, feat = x.shape
  n_edges = edge_index.shape[1]
  ncls = W3.shape[1]
  e_per_tile = n_edges // NW
  nblk = e_per_tile // K_EDGE
  assert e_per_tile * NW == n_edges and nblk * K_EDGE == e_per_tile
  assert n_nodes % NS == 0 and feat % LANES == 0 and n_nodes <= NPAD

  x = x.astype(jnp.float32)
  src = edge_index[0].astype(jnp.int32)
  dst = edge_index[1].astype(jnp.int32)
  src2d = src.reshape(NW, e_per_tile)
  dst2d = dst.reshape(NW, e_per_tile)
  src3d = src.reshape(NW, nblk, K_EDGE)
  dst3d = dst.reshape(NW, nblk, K_EDGE)
  b1r = b1.astype(jnp.float32).reshape(1, feat)
  b2r = b2.astype(jnp.float32).reshape(1, feat)
  b3r = b3.astype(jnp.float32).reshape(1, ncls)

  hist = _degree_kernel(src2d, dst2d, n_nodes, e_per_tile)
  inv = _norm_kernel(hist)
  h1 = _tc1(inv, x, W1.astype(jnp.float32), n_nodes, feat, 1000)
  p1 = _aggregate_kernel(h1, src3d, dst3d, n_nodes, nblk, feat)
  t2 = _tc_mid(p1, h1, inv, W2.astype(jnp.float32), b1r, n_nodes, feat, 1000)
  p2 = _aggregate_kernel(t2, src3d, dst3d, n_nodes, nblk, feat)
  out = _tc_out(p2, t2, inv, W3.astype(jnp.float32), b2r, b3r,
                n_nodes, feat, ncls, 1000)
  return out.astype(jnp.float64)
